# baseline (speedup 1.0000x reference)
import functools

import jax
import jax.numpy as jnp
from jax import lax
from jax.experimental import pallas as pl
from jax.experimental.pallas import tpu as pltpu
from jax.experimental.pallas import tpu_sc as plsc

_N = 10000
_E = 320000
_D = 128
_NC = 2
_NS = 16
_BLK = 125
_EPT = _E // _NS
_NBLK = _EPT // _BLK
_NP = 10112
_RPT = _NP // _NS
_CW = 16
_SB = 20
_NSB = _NBLK // _SB
_NPAIR = _SB // 2


def _sc_body(x_ps, x_gw, x_sw,
             s_psgw, d_psgw, s_gwps, d_gwps, s_pssw, d_pssw, s_swps, d_swps,
             z128, z16, ones_h,
             o_s_psgw, o_c_psgw, o_s_gwps, o_c_gwps,
             o_s_pssw, o_c_pssw, o_s_swps, o_c_swps,
             acc, cnt, sidx_sb, didx_sb, rows0, rows1,
             ones_v, semg0, semg1, sems0, sems1):
  core = lax.axis_index("c")
  sub = lax.axis_index("s")
  pltpu.sync_copy(ones_h, ones_v)
  rows = (rows0, rows1)
  semg = (semg0, semg1)
  sems = (sems0, sems1)

  def run(x_src, se2, de2, osum, ocnt):
    rbase = sub * _RPT
    pltpu.sync_copy(z128, acc.at[pl.ds(rbase, _RPT)])
    pltpu.sync_copy(z16, cnt.at[pl.ds(rbase, _RPT)])
    plsc.subcore_barrier()

    sbbase = sub * (_EPT // _BLK)

    def g_start(j, k):
      pltpu.make_async_copy(x_src.at[sidx_sb.at[j]], rows[k], semg[k]).start()

    def g_drain(j, k):
      pltpu.make_async_copy(x_src.at[sidx_sb.at[j]], rows[k], semg[k]).wait()

    def s_start(j, k):
      pltpu.make_async_copy(
          rows[k], acc.at[didx_sb.at[j]], sems[k]).start(add=True)
      pltpu.make_async_copy(
          ones_v, cnt.at[didx_sb.at[j]], sems[k]).start(add=True)

    def s_drain(j, k):
      pltpu.make_async_copy(rows[k], acc.at[didx_sb.at[j]], sems[k]).wait()
      pltpu.make_async_copy(ones_v, cnt.at[didx_sb.at[j]], sems[k]).wait()

    def pair(t, first, last):
      j0 = 2 * t
      j1 = j0 + 1
      if not first:
        s_drain(j1 - 2, 1)
      g_start(j1, 1)
      g_drain(j0, 0)
      s_start(j0, 0)
      g_drain(j1, 1)
      s_start(j1, 1)
      s_drain(j0, 0)
      if not last:
        g_start(j0 + 2, 0)
      else:
        s_drain(j1, 1)

    def super_block(s, carry):
      pltpu.sync_copy(se2.at[pl.ds(sbbase + s * _SB, _SB)], sidx_sb)
      pltpu.sync_copy(de2.at[pl.ds(sbbase + s * _SB, _SB)], didx_sb)
      g_start(0, 0)
      pair(0, True, False)
      lax.fori_loop(1, _NPAIR - 1, lambda t, c: (pair(t, False, False), c)[1],
                    carry)
      pair(_NPAIR - 1, False, True)
      return carry

    lax.fori_loop(0, _NSB, super_block, 0)
    plsc.subcore_barrier()

    pltpu.sync_copy(acc.at[pl.ds(rbase, _RPT)], osum.at[pl.ds(rbase, _RPT)])
    pltpu.sync_copy(cnt.at[pl.ds(rbase, _RPT)], ocnt.at[pl.ds(rbase, _RPT)])
    plsc.subcore_barrier()

  @pl.when(core == 0)
  def _():
    run(x_ps, s_psgw, d_psgw, o_s_psgw, o_c_psgw)
    run(x_gw, s_gwps, d_gwps, o_s_gwps, o_c_gwps)

  @pl.when(core == 1)
  def _():
    run(x_ps, s_pssw, d_pssw, o_s_pssw, o_c_pssw)
    run(x_sw, s_swps, d_swps, o_s_swps, o_c_swps)


_sum_t = jax.ShapeDtypeStruct((_NP, _D), jnp.float32)
_cnt_t = jax.ShapeDtypeStruct((_NP, _CW), jnp.float32)

_sc_agg = pl.kernel(
    _sc_body,
    out_type=[_sum_t, _cnt_t, _sum_t, _cnt_t, _sum_t, _cnt_t, _sum_t, _cnt_t],
    mesh=plsc.VectorSubcoreMesh(core_axis_name="c", subcore_axis_name="s",
                                num_cores=_NC, num_subcores=_NS),
    scratch_types=[
        pltpu.VMEM_SHARED((_NP, _D), jnp.float32),
        pltpu.VMEM_SHARED((_NP, _CW), jnp.float32),
        pltpu.VMEM((_SB, _BLK), jnp.int32),
        pltpu.VMEM((_SB, _BLK), jnp.int32),
        pltpu.VMEM((_BLK, _D), jnp.float32),
        pltpu.VMEM((_BLK, _D), jnp.float32),
        pltpu.VMEM((_BLK, _CW), jnp.float32),
        pltpu.SemaphoreType.DMA,
        pltpu.SemaphoreType.DMA,
        pltpu.SemaphoreType.DMA,
        pltpu.SemaphoreType.DMA,
    ],
    compiler_params=pltpu.CompilerParams(use_tc_tiling_on_sc=False),
)


def _dotT(a, w):
  return lax.dot_general(a, w, (((1,), (1,)), ((), ())),
                         preferred_element_type=jnp.float32)


def _tc_body(s_gwps, c_gwps, s_swps, c_swps, x_ps,
             s_psgw, c_psgw, x_gw, s_pssw, c_pssw, x_sw,
             wl_gwps, wr_gwps, blv_gwps, wl_swps, wr_swps, blv_swps,
             wl_psgw, wr_psgw, blv_psgw, wl_pssw, wr_pssw, blv_pssw,
             wg, bg, ag, ws, bs, asw,
             o_ps, o_gw, o_sw):
  def sage(s_ref, c_ref, x_ref, wl, wr, bl):
    cntc = jnp.maximum(c_ref[:, 0:1], 1.0)
    mean = s_ref[...] / cntc
    return _dotT(mean, wl[...]) + _dotT(x_ref[...], wr[...]) + bl[...]

  t_ps = (sage(s_gwps, c_gwps, x_ps, wl_gwps, wr_gwps, blv_gwps)
          + sage(s_swps, c_swps, x_ps, wl_swps, wr_swps, blv_swps))
  o_ps[...] = jnp.maximum(t_ps, 0.0)

  t_gw = jnp.maximum(sage(s_psgw, c_psgw, x_gw, wl_psgw, wr_psgw, blv_psgw), 0.0)
  z_gw = jnp.sum(t_gw * wg[...], axis=1, keepdims=True) + bg[...]
  o_gw[...] = jnp.where(z_gw >= 0.0, z_gw, ag[...] * z_gw)

  t_sw = jnp.maximum(sage(s_pssw, c_pssw, x_sw, wl_pssw, wr_pssw, blv_pssw), 0.0)
  z_sw = jnp.sum(t_sw * ws[...], axis=1, keepdims=True) + bs[...]
  o_sw[...] = jnp.where(z_sw >= 0.0, z_sw, asw[...] * z_sw)


_TCR = 1000


def _tc_epilogue(args):
  (s_gwps, c_gwps, s_swps, c_swps, x_ps, s_psgw, c_psgw, x_gw,
   s_pssw, c_pssw, x_sw, *rest) = args
  rowf = pl.BlockSpec((_TCR, _D), lambda i: (i, 0))
  rowc = pl.BlockSpec((_TCR, _CW), lambda i: (i, 0))
  mat = pl.BlockSpec((_D, _D), lambda i: (0, 0))
  vec = pl.BlockSpec((1, _D), lambda i: (0, 0))
  scl = pl.BlockSpec((1, 1), lambda i: (0, 0))
  in_specs = [rowf, rowc, rowf, rowc, rowf, rowf, rowc, rowf, rowf, rowc, rowf]
  in_specs += [mat, mat, vec, mat, mat, vec, mat, mat, vec, mat, mat, vec,
               vec, scl, scl, vec, scl, scl]
  return pl.pallas_call(
      _tc_body,
      grid=(_N // _TCR,),
      in_specs=in_specs,
      out_specs=[pl.BlockSpec((_TCR, _D), lambda i: (i, 0)),
                 pl.BlockSpec((_TCR, 1), lambda i: (i, 0)),
                 pl.BlockSpec((_TCR, 1), lambda i: (i, 0))],
      out_shape=[jax.ShapeDtypeStruct((_N, _D), jnp.float32),
                 jax.ShapeDtypeStruct((_N, 1), jnp.float32),
                 jax.ShapeDtypeStruct((_N, 1), jnp.float32)],
  )(*args)


@jax.jit
def kernel(x_pfas_sites, x_gw_wells, x_sw_stations,
           edge_index_ps_gw, edge_index_gw_ps, edge_index_ps_sw,
           edge_index_sw_ps,
           Wl_ps_gw, bl_ps_gw, Wr_ps_gw,
           Wl_gw_ps, bl_gw_ps, Wr_gw_ps,
           Wl_ps_sw, bl_ps_sw, Wr_ps_sw,
           Wl_sw_ps, bl_sw_ps, Wr_sw_ps,
           W_gw, b_gw, W_sw, b_sw, a_gw, a_sw):
  z128 = jnp.zeros((_RPT, _D), jnp.float32)
  z16 = jnp.zeros((_RPT, _CW), jnp.float32)
  ones_h = jnp.ones((_BLK, _CW), jnp.float32)

  def ei2(ei):
    return ei[0].reshape(_E // _BLK, _BLK), ei[1].reshape(_E // _BLK, _BLK)

  se_psgw, de_psgw = ei2(edge_index_ps_gw)
  se_gwps, de_gwps = ei2(edge_index_gw_ps)
  se_pssw, de_pssw = ei2(edge_index_ps_sw)
  se_swps, de_swps = ei2(edge_index_sw_ps)

  (sum_psgw, cnt_psgw, sum_gwps, cnt_gwps,
   sum_pssw, cnt_pssw, sum_swps, cnt_swps) = _sc_agg(
      x_pfas_sites, x_gw_wells, x_sw_stations,
      se_psgw, de_psgw, se_gwps, de_gwps,
      se_pssw, de_pssw, se_swps, de_swps,
      z128, z16, ones_h)

  out_ps, out_gw, out_sw = _tc_epilogue((
      sum_gwps, cnt_gwps, sum_swps, cnt_swps, x_pfas_sites,
      sum_psgw, cnt_psgw, x_gw_wells, sum_pssw, cnt_pssw, x_sw_stations,
      Wl_gw_ps, Wr_gw_ps, bl_gw_ps.reshape(1, _D),
      Wl_sw_ps, Wr_sw_ps, bl_sw_ps.reshape(1, _D),
      Wl_ps_gw, Wr_ps_gw, bl_ps_gw.reshape(1, _D),
      Wl_ps_sw, Wr_ps_sw, bl_ps_sw.reshape(1, _D),
      W_gw, b_gw.reshape(1, 1), a_gw.reshape(1, 1),
      W_sw, b_sw.reshape(1, 1), a_sw.reshape(1, 1)))
  return (out_ps, out_gw, out_sw)


# speedup vs baseline: 1.0581x; 1.0581x over previous
import functools

import jax
import jax.numpy as jnp
from jax import lax
from jax.experimental import pallas as pl
from jax.experimental.pallas import tpu as pltpu
from jax.experimental.pallas import tpu_sc as plsc

_N = 10000
_E = 320000
_D = 128
_NC = 2
_NS = 16
_BLK = 80
_EPT = _E // _NS
_NBLK = _EPT // _BLK
_NP = 10112
_RPT = _NP // _NS
_CW = 16
_SB = 50
_NSB = _NBLK // _SB
_NPAIR = _SB // 2


def _sc_body(x_ps, x_gw, x_sw,
             e_psgw, e_gwps, e_pssw, e_swps,
             z128, z16, ones_h,
             o_s_psgw, o_c_psgw, o_s_gwps, o_c_gwps,
             o_s_pssw, o_c_pssw, o_s_swps, o_c_swps,
             acc, cnt, sidx_sb, didx_sb, rows0, rows1,
             ones_v, semg0, semg1, sems0, sems1):
  core = lax.axis_index("c")
  sub = lax.axis_index("s")
  pltpu.sync_copy(ones_h, ones_v)
  rows = (rows0, rows1)
  semg = (semg0, semg1)
  sems = (sems0, sems1)

  def run(x_src, e2, osum, ocnt):
    rbase = sub * _RPT
    pltpu.sync_copy(z128, acc.at[pl.ds(rbase, _RPT)])
    pltpu.sync_copy(z16, cnt.at[pl.ds(rbase, _RPT)])
    plsc.subcore_barrier()

    sbbase = sub * (_EPT // _BLK)

    def g_start(j, k):
      pltpu.make_async_copy(x_src.at[sidx_sb.at[j]], rows[k], semg[k]).start()

    def g_drain(j, k):
      pltpu.make_async_copy(x_src.at[sidx_sb.at[j]], rows[k], semg[k]).wait()

    def s_start(j, k):
      pltpu.make_async_copy(
          rows[k], acc.at[didx_sb.at[j]], sems[k]).start(add=True)
      pltpu.make_async_copy(
          ones_v, cnt.at[didx_sb.at[j]], sems[k]).start(add=True)

    def s_drain(j, k):
      pltpu.make_async_copy(rows[k], acc.at[didx_sb.at[j]], sems[k]).wait()
      pltpu.make_async_copy(ones_v, cnt.at[didx_sb.at[j]], sems[k]).wait()

    def pair(t, first, last):
      j0 = 2 * t
      j1 = j0 + 1
      if not first:
        s_drain(j1 - 2, 1)
      g_start(j1, 1)
      g_drain(j0, 0)
      s_start(j0, 0)
      g_drain(j1, 1)
      s_start(j1, 1)
      s_drain(j0, 0)
      if not last:
        g_start(j0 + 2, 0)
      else:
        s_drain(j1, 1)

    def super_block(s, carry):
      pltpu.sync_copy(e2.at[pl.ds(sbbase + s * _SB, _SB)], sidx_sb)
      pltpu.sync_copy(e2.at[pl.ds(_E // _BLK + sbbase + s * _SB, _SB)],
                      didx_sb)
      g_start(0, 0)
      pair(0, True, False)
      lax.fori_loop(1, _NPAIR - 1, lambda t, c: (pair(t, False, False), c)[1],
                    carry)
      pair(_NPAIR - 1, False, True)
      return carry

    lax.fori_loop(0, _NSB, super_block, 0)
    plsc.subcore_barrier()

    pltpu.sync_copy(acc.at[pl.ds(rbase, _RPT)], osum.at[pl.ds(rbase, _RPT)])
    pltpu.sync_copy(cnt.at[pl.ds(rbase, _RPT)], ocnt.at[pl.ds(rbase, _RPT)])
    plsc.subcore_barrier()

  @pl.when(core == 0)
  def _():
    run(x_ps, e_psgw, o_s_psgw, o_c_psgw)
    run(x_gw, e_gwps, o_s_gwps, o_c_gwps)

  @pl.when(core == 1)
  def _():
    run(x_ps, e_pssw, o_s_pssw, o_c_pssw)
    run(x_sw, e_swps, o_s_swps, o_c_swps)


_sum_t = jax.ShapeDtypeStruct((_NP, _D), jnp.float32)
_cnt_t = jax.ShapeDtypeStruct((_NP, _CW), jnp.float32)

_sc_agg = pl.kernel(
    _sc_body,
    out_type=[_sum_t, _cnt_t, _sum_t, _cnt_t, _sum_t, _cnt_t, _sum_t, _cnt_t],
    mesh=plsc.VectorSubcoreMesh(core_axis_name="c", subcore_axis_name="s",
                                num_cores=_NC, num_subcores=_NS),
    scratch_types=[
        pltpu.VMEM_SHARED((_NP, _D), jnp.float32),
        pltpu.VMEM_SHARED((_NP, _CW), jnp.float32),
        pltpu.VMEM((_SB, _BLK), jnp.int32),
        pltpu.VMEM((_SB, _BLK), jnp.int32),
        pltpu.VMEM((_BLK, _D), jnp.float32),
        pltpu.VMEM((_BLK, _D), jnp.float32),
        pltpu.VMEM((_BLK, _CW), jnp.float32),
        pltpu.SemaphoreType.DMA,
        pltpu.SemaphoreType.DMA,
        pltpu.SemaphoreType.DMA,
        pltpu.SemaphoreType.DMA,
    ],
    compiler_params=pltpu.CompilerParams(use_tc_tiling_on_sc=False),
)


def _dotT(a, w):
  return lax.dot_general(a, w, (((1,), (1,)), ((), ())),
                         preferred_element_type=jnp.float32)


def _tc_body(s_gwps, c_gwps, s_swps, c_swps, x_ps,
             s_psgw, c_psgw, x_gw, s_pssw, c_pssw, x_sw,
             wl_gwps, wr_gwps, blv_gwps, wl_swps, wr_swps, blv_swps,
             wl_psgw, wr_psgw, blv_psgw, wl_pssw, wr_pssw, blv_pssw,
             wg, bg, ag, ws, bs, asw,
             o_ps, o_gw, o_sw):
  def sage(s_ref, c_ref, x_ref, wl, wr, bl):
    cntc = jnp.maximum(c_ref[:, 0:1], 1.0)
    mean = s_ref[...] / cntc
    return _dotT(mean, wl[...]) + _dotT(x_ref[...], wr[...]) + bl[...]

  t_ps = (sage(s_gwps, c_gwps, x_ps, wl_gwps, wr_gwps, blv_gwps)
          + sage(s_swps, c_swps, x_ps, wl_swps, wr_swps, blv_swps))
  o_ps[...] = jnp.maximum(t_ps, 0.0)

  t_gw = jnp.maximum(sage(s_psgw, c_psgw, x_gw, wl_psgw, wr_psgw, blv_psgw), 0.0)
  z_gw = jnp.sum(t_gw * wg[...], axis=1, keepdims=True) + bg[...]
  o_gw[...] = jnp.where(z_gw >= 0.0, z_gw, ag[...] * z_gw)

  t_sw = jnp.maximum(sage(s_pssw, c_pssw, x_sw, wl_pssw, wr_pssw, blv_pssw), 0.0)
  z_sw = jnp.sum(t_sw * ws[...], axis=1, keepdims=True) + bs[...]
  o_sw[...] = jnp.where(z_sw >= 0.0, z_sw, asw[...] * z_sw)


_TCR = 1000


def _tc_epilogue(args):
  (s_gwps, c_gwps, s_swps, c_swps, x_ps, s_psgw, c_psgw, x_gw,
   s_pssw, c_pssw, x_sw, *rest) = args
  rowf = pl.BlockSpec((_TCR, _D), lambda i: (i, 0))
  rowc = pl.BlockSpec((_TCR, _CW), lambda i: (i, 0))
  mat = pl.BlockSpec((_D, _D), lambda i: (0, 0))
  vec = pl.BlockSpec((1, _D), lambda i: (0, 0))
  scl = pl.BlockSpec((1, 1), lambda i: (0, 0))
  in_specs = [rowf, rowc, rowf, rowc, rowf, rowf, rowc, rowf, rowf, rowc, rowf]
  in_specs += [mat, mat, vec, mat, mat, vec, mat, mat, vec, mat, mat, vec,
               vec, scl, scl, vec, scl, scl]
  return pl.pallas_call(
      _tc_body,
      grid=(_N // _TCR,),
      in_specs=in_specs,
      out_specs=[pl.BlockSpec((_TCR, _D), lambda i: (i, 0)),
                 pl.BlockSpec((_TCR, 1), lambda i: (i, 0)),
                 pl.BlockSpec((_TCR, 1), lambda i: (i, 0))],
      out_shape=[jax.ShapeDtypeStruct((_N, _D), jnp.float32),
                 jax.ShapeDtypeStruct((_N, 1), jnp.float32),
                 jax.ShapeDtypeStruct((_N, 1), jnp.float32)],
  )(*args)


@jax.jit
def kernel(x_pfas_sites, x_gw_wells, x_sw_stations,
           edge_index_ps_gw, edge_index_gw_ps, edge_index_ps_sw,
           edge_index_sw_ps,
           Wl_ps_gw, bl_ps_gw, Wr_ps_gw,
           Wl_gw_ps, bl_gw_ps, Wr_gw_ps,
           Wl_ps_sw, bl_ps_sw, Wr_ps_sw,
           Wl_sw_ps, bl_sw_ps, Wr_sw_ps,
           W_gw, b_gw, W_sw, b_sw, a_gw, a_sw):
  z128 = jnp.zeros((_RPT, _D), jnp.float32)
  z16 = jnp.zeros((_RPT, _CW), jnp.float32)
  ones_h = jnp.ones((_BLK, _CW), jnp.float32)

  def ei2(ei):
    return ei.reshape(2 * _E // _BLK, _BLK)

  (sum_psgw, cnt_psgw, sum_gwps, cnt_gwps,
   sum_pssw, cnt_pssw, sum_swps, cnt_swps) = _sc_agg(
      x_pfas_sites, x_gw_wells, x_sw_stations,
      ei2(edge_index_ps_gw), ei2(edge_index_gw_ps),
      ei2(edge_index_ps_sw), ei2(edge_index_sw_ps),
      z128, z16, ones_h)

  out_ps, out_gw, out_sw = _tc_epilogue((
      sum_gwps, cnt_gwps, sum_swps, cnt_swps, x_pfas_sites,
      sum_psgw, cnt_psgw, x_gw_wells, sum_pssw, cnt_pssw, x_sw_stations,
      Wl_gw_ps, Wr_gw_ps, bl_gw_ps.reshape(1, _D),
      Wl_sw_ps, Wr_sw_ps, bl_sw_ps.reshape(1, _D),
      Wl_ps_gw, Wr_ps_gw, bl_ps_gw.reshape(1, _D),
      Wl_ps_sw, Wr_ps_sw, bl_ps_sw.reshape(1, _D),
      W_gw, b_gw.reshape(1, 1), a_gw.reshape(1, 1),
      W_sw, b_sw.reshape(1, 1), a_sw.reshape(1, 1)))
  return (out_ps, out_gw, out_sw)


# speedup vs baseline: 1.2167x; 1.1499x over previous
import functools

import jax
import jax.numpy as jnp
from jax import lax
from jax.experimental import pallas as pl
from jax.experimental.pallas import tpu as pltpu
from jax.experimental.pallas import tpu_sc as plsc

_N = 10000
_E = 320000
_D = 128
_NC = 2
_NS = 16
_BLK = 80
_EPT = _E // _NS
_NBLK = _EPT // _BLK
_NP = 10112
_RPT = _NP // _NS
_CW = 16
_SB = 50
_NSB = _NBLK // _SB
_NPAIR = _SB // 2


def _sc_body(x_ps, x_gw, x_sw,
             e_psgw, e_gwps, e_pssw, e_swps,
             z128, z16, ones_h,
             o_s_psgw, o_c_psgw, o_s_gwps, o_c_gwps,
             o_s_pssw, o_c_pssw, o_s_swps, o_c_swps,
             acc, cnt, sidx_sb, didx_sb, rows0, rows1,
             ones_v, semg0, semg1, semr0, semr1, semc0, semc1):
  core = lax.axis_index("c")
  sub = lax.axis_index("s")
  pltpu.sync_copy(ones_h, ones_v)
  rows = (rows0, rows1)
  semg = (semg0, semg1)
  semr = (semr0, semr1)
  semc = (semc0, semc1)

  def run(x_src, e2, osum, ocnt):
    rbase = sub * _RPT
    pltpu.sync_copy(z128, acc.at[pl.ds(rbase, _RPT)])
    pltpu.sync_copy(z16, cnt.at[pl.ds(rbase, _RPT)])
    plsc.subcore_barrier()

    sbbase = sub * (_EPT // _BLK)

    def g_start(j, k):
      pltpu.make_async_copy(x_src.at[sidx_sb.at[j]], rows[k], semg[k]).start()

    def g_drain(j, k):
      pltpu.make_async_copy(x_src.at[sidx_sb.at[j]], rows[k], semg[k]).wait()

    def s_start(j, k):
      pltpu.make_async_copy(
          rows[k], acc.at[didx_sb.at[j]], semr[k]).start(add=True)
      pltpu.make_async_copy(
          ones_v, cnt.at[didx_sb.at[j]], semc[k]).start(add=True)

    def s_drain(j, k):
      pltpu.make_async_copy(rows[k], acc.at[didx_sb.at[j]], semr[k]).wait()
      pltpu.make_async_copy(ones_v, cnt.at[didx_sb.at[j]], semc[k]).wait()

    def pair(t, first, last):
      j0 = 2 * t
      j1 = j0 + 1
      if not first:
        s_drain(j1 - 2, 1)
      g_start(j1, 1)
      g_drain(j0, 0)
      s_start(j0, 0)
      g_drain(j1, 1)
      s_start(j1, 1)
      s_drain(j0, 0)
      if not last:
        g_start(j0 + 2, 0)
      else:
        s_drain(j1, 1)

    def super_block(s, carry):
      pltpu.sync_copy(e2.at[pl.ds(sbbase + s * _SB, _SB)], sidx_sb)
      pltpu.sync_copy(e2.at[pl.ds(_E // _BLK + sbbase + s * _SB, _SB)],
                      didx_sb)
      g_start(0, 0)
      pair(0, True, False)
      lax.fori_loop(1, _NPAIR - 1, lambda t, c: (pair(t, False, False), c)[1],
                    carry)
      pair(_NPAIR - 1, False, True)
      return carry

    lax.fori_loop(0, _NSB, super_block, 0)
    plsc.subcore_barrier()

    pltpu.sync_copy(acc.at[pl.ds(rbase, _RPT)], osum.at[pl.ds(rbase, _RPT)])
    pltpu.sync_copy(cnt.at[pl.ds(rbase, _RPT)], ocnt.at[pl.ds(rbase, _RPT)])
    plsc.subcore_barrier()

  @pl.when(core == 0)
  def _():
    run(x_ps, e_psgw, o_s_psgw, o_c_psgw)
    run(x_gw, e_gwps, o_s_gwps, o_c_gwps)

  @pl.when(core == 1)
  def _():
    run(x_ps, e_pssw, o_s_pssw, o_c_pssw)
    run(x_sw, e_swps, o_s_swps, o_c_swps)


_sum_t = jax.ShapeDtypeStruct((_NP, _D), jnp.float32)
_cnt_t = jax.ShapeDtypeStruct((_NP, _CW), jnp.float32)

_sc_agg = pl.kernel(
    _sc_body,
    out_type=[_sum_t, _cnt_t, _sum_t, _cnt_t, _sum_t, _cnt_t, _sum_t, _cnt_t],
    mesh=plsc.VectorSubcoreMesh(core_axis_name="c", subcore_axis_name="s",
                                num_cores=_NC, num_subcores=_NS),
    scratch_types=[
        pltpu.VMEM_SHARED((_NP, _D), jnp.float32),
        pltpu.VMEM_SHARED((_NP, _CW), jnp.float32),
        pltpu.VMEM((_SB, _BLK), jnp.int32),
        pltpu.VMEM((_SB, _BLK), jnp.int32),
        pltpu.VMEM((_BLK, _D), jnp.float32),
        pltpu.VMEM((_BLK, _D), jnp.float32),
        pltpu.VMEM((_BLK, _CW), jnp.float32),
        pltpu.SemaphoreType.DMA,
        pltpu.SemaphoreType.DMA,
        pltpu.SemaphoreType.DMA,
        pltpu.SemaphoreType.DMA,
        pltpu.SemaphoreType.DMA,
        pltpu.SemaphoreType.DMA,
    ],
    compiler_params=pltpu.CompilerParams(use_tc_tiling_on_sc=False),
)


def _dotT(a, w):
  return lax.dot_general(a, w, (((1,), (1,)), ((), ())),
                         preferred_element_type=jnp.float32)


def _tc_body(s_gwps, c_gwps, s_swps, c_swps, x_ps,
             s_psgw, c_psgw, x_gw, s_pssw, c_pssw, x_sw,
             wl_gwps, wr_gwps, blv_gwps, wl_swps, wr_swps, blv_swps,
             wl_psgw, wr_psgw, blv_psgw, wl_pssw, wr_pssw, blv_pssw,
             wg, bg, ag, ws, bs, asw,
             o_ps, o_gw, o_sw):
  def sage(s_ref, c_ref, x_ref, wl, wr, bl):
    cntc = jnp.maximum(c_ref[:, 0:1], 1.0)
    mean = s_ref[...] / cntc
    return _dotT(mean, wl[...]) + _dotT(x_ref[...], wr[...]) + bl[...]

  t_ps = (sage(s_gwps, c_gwps, x_ps, wl_gwps, wr_gwps, blv_gwps)
          + sage(s_swps, c_swps, x_ps, wl_swps, wr_swps, blv_swps))
  o_ps[...] = jnp.maximum(t_ps, 0.0)

  t_gw = jnp.maximum(sage(s_psgw, c_psgw, x_gw, wl_psgw, wr_psgw, blv_psgw), 0.0)
  z_gw = jnp.sum(t_gw * wg[...], axis=1, keepdims=True) + bg[...]
  o_gw[...] = jnp.where(z_gw >= 0.0, z_gw, ag[...] * z_gw)

  t_sw = jnp.maximum(sage(s_pssw, c_pssw, x_sw, wl_pssw, wr_pssw, blv_pssw), 0.0)
  z_sw = jnp.sum(t_sw * ws[...], axis=1, keepdims=True) + bs[...]
  o_sw[...] = jnp.where(z_sw >= 0.0, z_sw, asw[...] * z_sw)


_TCR = 1000


def _tc_epilogue(args):
  (s_gwps, c_gwps, s_swps, c_swps, x_ps, s_psgw, c_psgw, x_gw,
   s_pssw, c_pssw, x_sw, *rest) = args
  rowf = pl.BlockSpec((_TCR, _D), lambda i: (i, 0))
  rowc = pl.BlockSpec((_TCR, _CW), lambda i: (i, 0))
  mat = pl.BlockSpec((_D, _D), lambda i: (0, 0))
  vec = pl.BlockSpec((1, _D), lambda i: (0, 0))
  scl = pl.BlockSpec((1, 1), lambda i: (0, 0))
  in_specs = [rowf, rowc, rowf, rowc, rowf, rowf, rowc, rowf, rowf, rowc, rowf]
  in_specs += [mat, mat, vec, mat, mat, vec, mat, mat, vec, mat, mat, vec,
               vec, scl, scl, vec, scl, scl]
  return pl.pallas_call(
      _tc_body,
      grid=(_N // _TCR,),
      in_specs=in_specs,
      out_specs=[pl.BlockSpec((_TCR, _D), lambda i: (i, 0)),
                 pl.BlockSpec((_TCR, 1), lambda i: (i, 0)),
                 pl.BlockSpec((_TCR, 1), lambda i: (i, 0))],
      out_shape=[jax.ShapeDtypeStruct((_N, _D), jnp.float32),
                 jax.ShapeDtypeStruct((_N, 1), jnp.float32),
                 jax.ShapeDtypeStruct((_N, 1), jnp.float32)],
  )(*args)


@jax.jit
def kernel(x_pfas_sites, x_gw_wells, x_sw_stations,
           edge_index_ps_gw, edge_index_gw_ps, edge_index_ps_sw,
           edge_index_sw_ps,
           Wl_ps_gw, bl_ps_gw, Wr_ps_gw,
           Wl_gw_ps, bl_gw_ps, Wr_gw_ps,
           Wl_ps_sw, bl_ps_sw, Wr_ps_sw,
           Wl_sw_ps, bl_sw_ps, Wr_sw_ps,
           W_gw, b_gw, W_sw, b_sw, a_gw, a_sw):
  z128 = jnp.zeros((_RPT, _D), jnp.float32)
  z16 = jnp.zeros((_RPT, _CW), jnp.float32)
  ones_h = jnp.ones((_BLK, _CW), jnp.float32)

  def ei2(ei):
    return ei.reshape(2 * _E // _BLK, _BLK)

  (sum_psgw, cnt_psgw, sum_gwps, cnt_gwps,
   sum_pssw, cnt_pssw, sum_swps, cnt_swps) = _sc_agg(
      x_pfas_sites, x_gw_wells, x_sw_stations,
      ei2(edge_index_ps_gw), ei2(edge_index_gw_ps),
      ei2(edge_index_ps_sw), ei2(edge_index_sw_ps),
      z128, z16, ones_h)

  out_ps, out_gw, out_sw = _tc_epilogue((
      sum_gwps, cnt_gwps, sum_swps, cnt_swps, x_pfas_sites,
      sum_psgw, cnt_psgw, x_gw_wells, sum_pssw, cnt_pssw, x_sw_stations,
      Wl_gw_ps, Wr_gw_ps, bl_gw_ps.reshape(1, _D),
      Wl_sw_ps, Wr_sw_ps, bl_sw_ps.reshape(1, _D),
      Wl_ps_gw, Wr_ps_gw, bl_ps_gw.reshape(1, _D),
      Wl_ps_sw, Wr_ps_sw, bl_ps_sw.reshape(1, _D),
      W_gw, b_gw.reshape(1, 1), a_gw.reshape(1, 1),
      W_sw, b_sw.reshape(1, 1), a_sw.reshape(1, 1)))
  return (out_ps, out_gw, out_sw)
